# Initial kernel scaffold; baseline (speedup 1.0000x reference)
import jax
import jax.numpy as jnp
from jax.experimental import pallas as pl


def kernel(mem, idx, val, sample_idx):
    raise NotImplementedError("write your pallas kernel here")


# speedup vs baseline: 3.0201x; 3.0201x over previous
import jax
import jax.numpy as jnp
from jax import lax
from jax.experimental import pallas as pl
from jax.experimental.pallas import tpu as pltpu
from jax.experimental.pallas import tpu_sc as plsc

CAP = 1_000_000
D = 64
B = 16384
NC = 2
NS = 16
NW = NC * NS
SPT = B // NW
RI = 128
NRS = SPT // RI
NRB = B // RI
RPT = NRB // NS

MSET_CH = 3136
TAG_TILE = 20 * MSET_CH
TAG_CAP = TAG_TILE * NS
CNT_TILE = 5 * MSET_CH
CNT_CAP = CNT_TILE * NS
SCH = NRB // 16


def _tile_body(mem_hbm, idx_hbm, val_hbm, samp_hbm, out_hbm,
               tag_sh, cnt_sh, idx_sh, mset_v, samp_v, t_v, c_v, rows_v,
               vtmp_v, hidx_v, tfx_v, idx_own, scan_v, jv_v, cs_v, cw_v,
               sem):
    cid = lax.axis_index("c")
    sid = lax.axis_index("s")
    wid = sid * NC + cid
    base = wid * SPT
    iot = lax.iota(jnp.int32, 16)

    pltpu.sync_copy(samp_hbm.at[wid], samp_v)
    row_cps = [
        pltpu.async_copy(mem_hbm.at[samp_v.at[r]], rows_v.at[r], sem)
        for r in range(NRS)
    ]
    pltpu.sync_copy(idx_hbm.at[pl.ds(sid * RPT, RPT)], idx_own)
    pltpu.sync_copy(idx_own, idx_sh.at[pl.ds(sid * RPT, RPT)])

    def prep(k, x):
        row = sid * RPT + k
        for cb in range(8):
            blk = pl.ds(cb * 16, 16)
            ii = idx_own[k, blk]
            jv_v[k, blk] = row * 128 + cb * 16 + iot
            cs_v[k, blk] = jnp.int32(1) << ((ii & 3) * 8)
            cw_v[k, blk] = ii >> 2
        return x
    lax.fori_loop(0, RPT, prep, 0)

    def fill_m1(i, x):
        mset_v[pl.ds(i * 16, 16)] = jnp.full((16,), -1, jnp.int32)
        return x
    lax.fori_loop(0, MSET_CH // 16, fill_m1, 0)
    for k in range(TAG_TILE // MSET_CH):
        pltpu.sync_copy(
            mset_v, tag_sh.at[pl.ds(sid * TAG_TILE + k * MSET_CH, MSET_CH)])

    def fill_z(i, x):
        mset_v[pl.ds(i * 16, 16)] = jnp.full((16,), 0, jnp.int32)
        return x
    lax.fori_loop(0, MSET_CH // 16, fill_z, 0)
    for k in range(CNT_TILE // MSET_CH):
        pltpu.sync_copy(
            mset_v, cnt_sh.at[pl.ds(sid * CNT_TILE + k * MSET_CH, MSET_CH)])

    plsc.subcore_barrier()

    def scat(k, x):
        pltpu.sync_copy(jv_v.at[k], tag_sh.at[idx_own.at[k]])
        pltpu.sync_copy(cs_v.at[k], cnt_sh.at[cw_v.at[k]], add=True)
        return x
    lax.fori_loop(0, RPT, scat, 0)

    plsc.subcore_barrier()

    def swords(r, x):
        for cb in range(8):
            blk = pl.ds(cb * 16, 16)
            cw_v[RPT + r, blk] = samp_v[r, blk] >> 2
        return x
    lax.fori_loop(0, NRS, swords, 0)
    for r in range(NRS):
        pltpu.sync_copy(tag_sh.at[samp_v.at[r]], t_v.at[r])
        pltpu.sync_copy(cnt_sh.at[cw_v.at[RPT + r]], c_v.at[r])
    for cp in row_cps:
        cp.wait()

    def merge(c, x):
        r = c // 8
        col = (c % 8) * 16
        sb = samp_v[r, pl.ds(col, 16)]
        tb = t_v[r, pl.ds(col, 16)]
        cb_w = c_v[r, pl.ds(col, 16)]
        fld = (lax.shift_right_logical(cb_w, (sb & 3) * 8)) & 255
        hit = tb >= 0
        multi = hit & (fld != 1)
        hit_i = jnp.where(hit, 1, 0).astype(jnp.int32)
        multi_i = jnp.where(multi, 1, 0).astype(jnp.int32)
        anyh = jnp.max(hit_i)

        @pl.when(anyh > 0)
        def _():
            tfx_v[...] = tb

            @pl.when(jnp.max(multi_i) > 0)
            def _():
                for n in range(16):
                    @pl.when(multi_i[n] > 0)
                    def _(n=n):
                        pv16 = lax.broadcast(sb[n], (16,))

                        def chunk(ch, best):
                            pltpu.sync_copy(
                                idx_sh.at[pl.ds(ch * SCH, SCH)], scan_v)

                            def sbod(rr, best):
                                for cb2 in range(8):
                                    blk2 = pl.ds(cb2 * 16, 16)
                                    m = scan_v[rr, blk2] == pv16
                                    jj = ((ch * SCH + rr) * 128
                                          + cb2 * 16 + iot)
                                    best = jnp.maximum(
                                        best, jnp.where(m, jj, -1))
                                return best
                            return lax.fori_loop(0, SCH, sbod, best)
                        best = lax.fori_loop(
                            0, 16, chunk, jnp.full((16,), -1, jnp.int32))
                        jn = jnp.max(best)
                        tfx_v[...] = jnp.where(
                            iot == n, lax.broadcast(jn, (16,)), tfx_v[...])

            hidx_v[...] = jnp.where(hit, tfx_v[...], base + c * 16 + iot)
            pltpu.sync_copy(val_hbm.at[hidx_v], vtmp_v)
            for n in range(16):
                pv = lax.broadcast(hit_i[n], (16,)) > 0
                for d in range(4):
                    dblk = pl.ds(d * 16, 16)
                    rows_v[r, col + n, dblk] = jnp.where(
                        pv, vtmp_v[n, dblk], rows_v[r, col + n, dblk])
        return x
    lax.fori_loop(0, SPT // 16, merge, 0)

    for r in range(NRS):
        pltpu.sync_copy(rows_v.at[r], out_hbm.at[pl.ds(base + r * RI, RI)])


@jax.jit
def kernel(mem, idx, val, sample_idx):
    idx2 = idx.reshape(NRB, RI)
    samp = sample_idx.reshape(NW, NRS, RI)
    mesh = plsc.VectorSubcoreMesh(core_axis_name="c", subcore_axis_name="s")
    run = pl.kernel(
        _tile_body,
        out_type=jax.ShapeDtypeStruct((B, D), jnp.float32),
        mesh=mesh,
        scratch_types=[
            pltpu.VMEM_SHARED((TAG_CAP,), jnp.int32),
            pltpu.VMEM_SHARED((CNT_CAP,), jnp.int32),
            pltpu.VMEM_SHARED((NRB, RI), jnp.int32),
            pltpu.VMEM((MSET_CH,), jnp.int32),
            pltpu.VMEM((NRS, RI), jnp.int32),
            pltpu.VMEM((NRS, RI), jnp.int32),
            pltpu.VMEM((NRS, RI), jnp.int32),
            pltpu.VMEM((NRS, RI, D), jnp.float32),
            pltpu.VMEM((16, D), jnp.float32),
            pltpu.VMEM((16,), jnp.int32),
            pltpu.VMEM((16,), jnp.int32),
            pltpu.VMEM((RPT, RI), jnp.int32),
            pltpu.VMEM((SCH, RI), jnp.int32),
            pltpu.VMEM((RPT, RI), jnp.int32),
            pltpu.VMEM((RPT, RI), jnp.int32),
            pltpu.VMEM((RPT + NRS, RI), jnp.int32),
            pltpu.SemaphoreType.DMA,
        ],
        compiler_params=pltpu.CompilerParams(
            needs_layout_passes=False, use_tc_tiling_on_sc=False),
    )
    return run(mem, idx2, val, samp)
